# baseline (speedup 1.0000x reference)
import jax
import jax.numpy as jnp
from jax import lax
from jax.experimental import pallas as pl
from jax.experimental.pallas import tpu as pltpu
from jax.experimental.pallas import tpu_sc as plsc

L = 16
D = 32
B = 16384
NC = 2
NS = 16
NW = NC * NS
BPW = B // NW
WAVE = 2
WPT = BPW // WAVE
DEPTH = 7
WPG = L // WAVE


def _sc_body(uid_hbm, iid_hbm, ut_hbm, it_hbm, out_hbm,
             uidx_v, iidx_v, ubuf_v, ibuf_v, out_v, usem, isem):
    wid = lax.axis_index("s") * NC + lax.axis_index("c")
    base = wid * BPW

    pltpu.sync_copy(uid_hbm.at[pl.ds(base, BPW)], uidx_v)
    pltpu.sync_copy(iid_hbm.at[pl.ds(base, BPW)], iidx_v)

    iota = lax.iota(jnp.int32, L)

    def vecs(w):
        g = (w // WPG) * L
        return uidx_v[pl.ds(g, L)], iidx_v[pl.ds(g, L)]

    def extract(vec, l):
        return lax.reduce_max(jnp.where(iota == l, vec, 0), (0,))

    def fire(w):
        uvec, ivec = vecs(w)
        p = w % DEPTH
        for s in range(WAVE):
            l = (w % WPG) * WAVE + s
            for vec, tab, buf, sem in ((uvec, ut_hbm, ubuf_v, usem),
                                       (ivec, it_hbm, ibuf_v, isem)):
                sid = extract(vec, l)
                jb = pl.multiple_of((sid >> 7) << 7, 128)
                pltpu.async_copy(tab.at[:, pl.ds(jb, 128)], buf.at[p, s], sem)

    def wait_wave():
        for s in range(WAVE):
            pltpu.make_async_copy(ut_hbm.at[:, pl.ds(0, 128)],
                                  ubuf_v.at[0, s], usem).wait()
            pltpu.make_async_copy(it_hbm.at[:, pl.ds(0, 128)],
                                  ibuf_v.at[0, s], isem).wait()

    d_lo = iota
    d_hi = iota + L

    def compute(w):
        uvec, ivec = vecs(w)
        p = w % DEPTH
        pb = jnp.full((L,), 0, jnp.int32) + p
        for s in range(WAVE):
            l = (w % WPG) * WAVE + s
            sb = jnp.full((L,), s, jnp.int32)
            usid = extract(uvec, l)
            isid = extract(ivec, l)
            ulane = jnp.full((L,), 0, jnp.int32) + (usid & 127)
            ilane = jnp.full((L,), 0, jnp.int32) + (isid & 127)
            u_lo = plsc.load_gather(ubuf_v, [pb, sb, d_lo, ulane])
            u_hi = plsc.load_gather(ubuf_v, [pb, sb, d_hi, ulane])
            i_lo = plsc.load_gather(ibuf_v, [pb, sb, d_lo, ilane])
            i_hi = plsc.load_gather(ibuf_v, [pb, sb, d_hi, ilane])
            prod = u_lo * i_lo + u_hi * i_hi
            score = lax.reduce_sum(prod, (0,))
            k = jnp.full((L,), 0, jnp.int32) + (w * WAVE + s)
            plsc.store_scatter(out_v, [k],
                               jnp.full((L,), 0.0, jnp.float32) + score,
                               mask=iota == 0)

    for w0 in range(DEPTH - 1):
        fire(w0)

    def body(w, carry):
        @pl.when(w + DEPTH - 1 < WPT)
        def _():
            fire(w + DEPTH - 1)
        wait_wave()
        compute(w)
        return carry

    lax.fori_loop(0, WPT, body, 0)

    pltpu.sync_copy(out_v, out_hbm.at[pl.ds(base, BPW)])


@jax.jit
def _run(user_ids, item_ids, user_table_t, item_table_t):
    k = pl.kernel(
        _sc_body,
        out_type=jax.ShapeDtypeStruct((B,), jnp.float32),
        mesh=plsc.VectorSubcoreMesh(core_axis_name="c", subcore_axis_name="s"),
        compiler_params=pltpu.CompilerParams(needs_layout_passes=False),
        scratch_types=[
            pltpu.VMEM((BPW,), jnp.int32),
            pltpu.VMEM((BPW,), jnp.int32),
            pltpu.VMEM((DEPTH, WAVE, D, 128), jnp.float32),
            pltpu.VMEM((DEPTH, WAVE, D, 128), jnp.float32),
            pltpu.VMEM((BPW,), jnp.float32),
            pltpu.SemaphoreType.DMA,
            pltpu.SemaphoreType.DMA,
        ],
    )
    return k(user_ids, item_ids, user_table_t, item_table_t)


def kernel(user_ids, item_ids, user_table, item_table):
    return _run(user_ids, item_ids, user_table.T, item_table.T)
